# baseline (speedup 1.0000x reference)
import jax
import jax.numpy as jnp
import numpy as np
from jax import lax
from jax.experimental import pallas as pl
from jax.experimental.pallas import tpu as pltpu
from jax.experimental.pallas import tpu_sc as plsc

B, L, V, D = 4096, 50, 100002, 300
C = 128
NW = 32
BPW = B // NW
CROWS = 128
NCHUNK = BPW * L // CROWS
NPAIR = NCHUNK // 2

_SIDX = (np.arange(16)[:, None] * BPW
         + np.arange(BPW * L) // L).reshape(16, NCHUNK, CROWS).astype(np.int32)


def _pool_body(x_hbm, z_hbm, sidx_hbm, tab_hbm, out_hbm,
               idx_v, sidx_v, buf_a, buf_b, acc_sh, sem_a, sem_b):
    c = lax.axis_index("c")
    s = lax.axis_index("s")
    wid = s * 2 + c
    abase = s * BPW

    pltpu.sync_copy(z_hbm, acc_sh.at[pl.ds(abase, BPW)])
    pltpu.sync_copy(x_hbm.at[pl.ds(wid * NCHUNK * CROWS, NCHUNK * CROWS)],
                    idx_v)
    pltpu.sync_copy(sidx_hbm.at[s], sidx_v)

    def fire(ci, buf, sem):
        pltpu.async_copy(
            tab_hbm.at[idx_v.at[pl.ds(ci * CROWS, CROWS)]], buf, sem)

    def drain_scatter(ci, buf, sem):
        pltpu.make_async_copy(
            tab_hbm.at[idx_v.at[pl.ds(ci * CROWS, CROWS)]], buf, sem).wait()
        pltpu.sync_copy(buf, acc_sh.at[sidx_v.at[ci]], add=True)

    fire(0, buf_a, sem_a)
    fire(1, buf_b, sem_b)

    def pair_body(p, carry):
        ci = 2 * p
        drain_scatter(ci, buf_a, sem_a)
        fire(ci + 2, buf_a, sem_a)
        drain_scatter(ci + 1, buf_b, sem_b)
        fire(ci + 3, buf_b, sem_b)
        return carry

    lax.fori_loop(0, NPAIR - 1, pair_body, 0)

    ci = NCHUNK - 2
    drain_scatter(ci, buf_a, sem_a)
    drain_scatter(ci + 1, buf_b, sem_b)

    plsc.subcore_barrier()
    pltpu.sync_copy(acc_sh.at[pl.ds(abase, BPW)],
                    out_hbm.at[pl.ds(wid * BPW, BPW)])


def _pool_slice(xi, tab):
    mesh = plsc.VectorSubcoreMesh(core_axis_name="c", subcore_axis_name="s")
    kern = pl.kernel(
        _pool_body,
        out_type=jax.ShapeDtypeStruct((B, C), jnp.float32),
        mesh=mesh,
        compiler_params=pltpu.CompilerParams(use_tc_tiling_on_sc=True),
        scratch_types=[
            pltpu.VMEM((NCHUNK * CROWS,), jnp.int32),
            pltpu.VMEM((NCHUNK, CROWS), jnp.int32),
            pltpu.VMEM((CROWS, C), jnp.float32),
            pltpu.VMEM((CROWS, C), jnp.float32),
            pltpu.VMEM_SHARED((B // 2, C), jnp.float32),
            pltpu.SemaphoreType.DMA,
            pltpu.SemaphoreType.DMA,
        ],
    )
    zeros = jnp.zeros((BPW, C), jnp.float32)
    return kern(xi, zeros, _SIDX, tab)


def _slice_body(tt_ref, o0_ref, o1_ref, o2_ref):
    xt = tt_ref[...].T
    o0_ref[...] = xt[:, 0:C]
    o1_ref[...] = xt[:, C:2 * C]
    o2_ref[...] = xt[:, D - C:D]


def _slices(tableT):
    BN = 512
    return pl.pallas_call(
        _slice_body,
        grid=((V + BN - 1) // BN,),
        in_specs=[pl.BlockSpec((D, BN), lambda i: (0, i))],
        out_specs=[pl.BlockSpec((BN, C), lambda i: (i, 0))] * 3,
        out_shape=[jax.ShapeDtypeStruct((V, C), jnp.float32)] * 3,
    )(tableT)


def _mlp_body(x0_ref, x1_ref, x2_ref, w1a_ref, w1b_ref, w1c_ref, b1_ref,
              w2_ref, b2_ref, w3_ref, b3_ref, o_ref):
    h = (jnp.dot(x0_ref[...], w1a_ref[...], preferred_element_type=jnp.float32)
         + jnp.dot(x1_ref[...], w1b_ref[...],
                   preferred_element_type=jnp.float32)
         + jnp.dot(x2_ref[...], w1c_ref[...],
                   preferred_element_type=jnp.float32))
    h = h * (1.0 / L) + b1_ref[...]
    h = jnp.where(h >= 0, h, 0.01 * h)
    h = jnp.dot(h, w2_ref[...], preferred_element_type=jnp.float32)
    h = h + b2_ref[...]
    h = jnp.where(h >= 0, h, 0.01 * h)
    h = jnp.dot(h, w3_ref[...], preferred_element_type=jnp.float32)
    o_ref[...] = h + b3_ref[...]


def _mlp(x0, x1, x2, w1a, w1b, w1c, b1, W2, b2, W3, b3):
    blk = 512
    xspec = pl.BlockSpec((blk, C), lambda i: (i, 0))
    w1spec = pl.BlockSpec((C, D), lambda i: (0, 0))
    wspec = pl.BlockSpec((D, D), lambda i: (0, 0))
    bspec = pl.BlockSpec((1, D), lambda i: (0, 0))
    return pl.pallas_call(
        _mlp_body,
        grid=(B // blk,),
        in_specs=[xspec, xspec, xspec, w1spec, w1spec, w1spec,
                  bspec, wspec, bspec, wspec, bspec],
        out_specs=pl.BlockSpec((blk, D), lambda i: (i, 0)),
        out_shape=jax.ShapeDtypeStruct((B, D), jnp.float32),
    )(x0, x1, x2, w1a, w1b, w1c, b1.reshape(1, D), W2, b2.reshape(1, D),
      W3, b3.reshape(1, D))


def kernel(x, table, W1, b1, W2, b2, W3, b3):
    xi = x.astype(jnp.int32).reshape(NW * NCHUNK * CROWS)
    t0, t1, t2 = _slices(table.T)
    w1a = W1[0:C]
    w1b = W1[C:2 * C]
    w1c = jnp.concatenate(
        [jnp.zeros((2 * C - (D - C), D), jnp.float32), W1[2 * C:D]], axis=0)
    x0 = _pool_slice(xi, t0)
    x1 = _pool_slice(xi, t1)
    x2 = _pool_slice(xi, t2)
    return _mlp(x0, x1, x2, w1a, w1b, w1c, b1, W2, b2, W3, b3)


# speedup vs baseline: 1.0111x; 1.0111x over previous
import jax
import jax.numpy as jnp
import numpy as np
from jax import lax
from jax.experimental import pallas as pl
from jax.experimental.pallas import tpu as pltpu
from jax.experimental.pallas import tpu_sc as plsc

B, L, V, D = 4096, 50, 100002, 300
C = 128
NW = 32
BPW = B // NW
CROWS = 128
NCHUNK = BPW * L // CROWS
NPAIR = NCHUNK // 2

_SIDX = (np.arange(16)[:, None] * BPW
         + np.arange(BPW * L) // L).reshape(16, NCHUNK, CROWS).astype(np.int32)


def _pool_body(x_hbm, z_hbm, sidx_hbm, tab_hbm, out_hbm,
               idx_v, sidx_v, buf_a, buf_b, acc_sh, sem_a, sem_b):
    c = lax.axis_index("c")
    s = lax.axis_index("s")
    wid = s * 2 + c
    abase = s * BPW

    pltpu.sync_copy(z_hbm, acc_sh.at[pl.ds(abase, BPW)])
    pltpu.sync_copy(x_hbm.at[pl.ds(wid * NCHUNK * CROWS, NCHUNK * CROWS)],
                    idx_v)
    pltpu.sync_copy(sidx_hbm.at[s], sidx_v)

    def fire(ci, buf, sem):
        pltpu.async_copy(
            tab_hbm.at[idx_v.at[pl.ds(ci * CROWS, CROWS)]], buf, sem)

    def drain_scatter(ci, buf, sem):
        pltpu.make_async_copy(
            tab_hbm.at[idx_v.at[pl.ds(ci * CROWS, CROWS)]], buf, sem).wait()
        pltpu.sync_copy(buf, acc_sh.at[sidx_v.at[ci]], add=True)

    fire(0, buf_a, sem_a)
    fire(1, buf_b, sem_b)

    def pair_body(p, carry):
        ci = 2 * p
        drain_scatter(ci, buf_a, sem_a)
        fire(ci + 2, buf_a, sem_a)
        drain_scatter(ci + 1, buf_b, sem_b)
        fire(ci + 3, buf_b, sem_b)
        return carry

    lax.fori_loop(0, NPAIR - 1, pair_body, 0)

    ci = NCHUNK - 2
    drain_scatter(ci, buf_a, sem_a)
    drain_scatter(ci + 1, buf_b, sem_b)

    plsc.subcore_barrier()
    pltpu.sync_copy(acc_sh.at[pl.ds(abase, BPW)],
                    out_hbm.at[pl.ds(wid * BPW, BPW)])


def _pool_slice(xi, tab):
    mesh = plsc.VectorSubcoreMesh(core_axis_name="c", subcore_axis_name="s")
    kern = pl.kernel(
        _pool_body,
        out_type=jax.ShapeDtypeStruct((B, C), jnp.float32),
        mesh=mesh,
        compiler_params=pltpu.CompilerParams(use_tc_tiling_on_sc=True),
        scratch_types=[
            pltpu.VMEM((NCHUNK * CROWS,), jnp.int32),
            pltpu.VMEM((NCHUNK, CROWS), jnp.int32),
            pltpu.VMEM((CROWS, C), jnp.float32),
            pltpu.VMEM((CROWS, C), jnp.float32),
            pltpu.VMEM_SHARED((B // 2, C), jnp.float32),
            pltpu.SemaphoreType.DMA,
            pltpu.SemaphoreType.DMA,
        ],
    )
    zeros = jnp.zeros((BPW, C), jnp.float32)
    return kern(xi, zeros, _SIDX, tab)


def _slice_body(tt_ref, o0_ref, o1_ref):
    xt = tt_ref[...].T
    o0_ref[...] = xt[:, 0:C]
    o1_ref[...] = xt[:, C:2 * C]


def _slices01(tableT):
    BN = 512
    return pl.pallas_call(
        _slice_body,
        grid=((V + BN - 1) // BN,),
        in_specs=[pl.BlockSpec((2 * C, BN), lambda i: (0, i))],
        out_specs=[pl.BlockSpec((BN, C), lambda i: (i, 0))] * 2,
        out_shape=[jax.ShapeDtypeStruct((V, C), jnp.float32)] * 2,
    )(tableT)


def _mlp_body(x0_ref, x1_ref, x2_ref, w1a_ref, w1b_ref, w1c_ref, b1_ref,
              w2_ref, b2_ref, w3_ref, b3_ref, o_ref):
    h = (jnp.dot(x0_ref[...], w1a_ref[...], preferred_element_type=jnp.float32)
         + jnp.dot(x1_ref[...], w1b_ref[...],
                   preferred_element_type=jnp.float32)
         + jnp.dot(x2_ref[...], w1c_ref[...],
                   preferred_element_type=jnp.float32))
    h = h * (1.0 / L) + b1_ref[...]
    h = jnp.where(h >= 0, h, 0.01 * h)
    h = jnp.dot(h, w2_ref[...], preferred_element_type=jnp.float32)
    h = h + b2_ref[...]
    h = jnp.where(h >= 0, h, 0.01 * h)
    h = jnp.dot(h, w3_ref[...], preferred_element_type=jnp.float32)
    o_ref[...] = h + b3_ref[...]


def _mlp(x0, x1, x2, w1a, w1b, w1c, b1, W2, b2, W3, b3):
    blk = 512
    xspec = pl.BlockSpec((blk, C), lambda i: (i, 0))
    w1spec = pl.BlockSpec((C, D), lambda i: (0, 0))
    wspec = pl.BlockSpec((D, D), lambda i: (0, 0))
    bspec = pl.BlockSpec((1, D), lambda i: (0, 0))
    return pl.pallas_call(
        _mlp_body,
        grid=(B // blk,),
        in_specs=[xspec, xspec, xspec, w1spec, w1spec, w1spec,
                  bspec, wspec, bspec, wspec, bspec],
        out_specs=pl.BlockSpec((blk, D), lambda i: (i, 0)),
        out_shape=jax.ShapeDtypeStruct((B, D), jnp.float32),
    )(x0, x1, x2, w1a, w1b, w1c, b1.reshape(1, D), W2, b2.reshape(1, D),
      W3, b3.reshape(1, D))


def kernel(x, table, W1, b1, W2, b2, W3, b3):
    xi = x.astype(jnp.int32).reshape(NW * NCHUNK * CROWS)
    t0, t1 = _slices01(table.T)
    t2 = table[:, D - C:D]
    w1a = W1[0:C]
    w1b = W1[C:2 * C]
    w1c = jnp.concatenate(
        [jnp.zeros((2 * C - (D - C), D), jnp.float32), W1[2 * C:D]], axis=0)
    x0 = _pool_slice(xi, t0)
    x1 = _pool_slice(xi, t1)
    x2 = _pool_slice(xi, t2)
    return _mlp(x0, x1, x2, w1a, w1b, w1c, b1, W2, b2, W3, b3)


# speedup vs baseline: 1.0601x; 1.0485x over previous
import jax
import jax.numpy as jnp
import numpy as np
from jax import lax
from jax.experimental import pallas as pl
from jax.experimental.pallas import tpu as pltpu
from jax.experimental.pallas import tpu_sc as plsc

B, L, V, D = 4096, 50, 100002, 300
C = 128
NW = 32
BPW = B // NW
CROWS = 128
NCHUNK = BPW * L // CROWS
NPAIR = NCHUNK // 2

_SIDX = (np.arange(16)[:, None] * BPW
         + np.arange(BPW * L) // L).reshape(16, NCHUNK, CROWS).astype(np.int32)


def _pool_body(x_hbm, z_hbm, sidx_hbm, tab_hbm, out_hbm,
               idx_v, sidx_v, buf_a, buf_b, acc_sh, sem_a, sem_b):
    c = lax.axis_index("c")
    s = lax.axis_index("s")
    wid = s * 2 + c
    abase = s * BPW

    pltpu.sync_copy(z_hbm, acc_sh.at[pl.ds(abase, BPW)])
    pltpu.sync_copy(x_hbm.at[pl.ds(wid * NCHUNK * CROWS, NCHUNK * CROWS)],
                    idx_v)
    pltpu.sync_copy(sidx_hbm.at[s], sidx_v)

    def fire(ci, buf, sem):
        pltpu.async_copy(
            tab_hbm.at[idx_v.at[pl.ds(ci * CROWS, CROWS)]], buf, sem)

    def drain_scatter(ci, buf, sem):
        pltpu.make_async_copy(
            tab_hbm.at[idx_v.at[pl.ds(ci * CROWS, CROWS)]], buf, sem).wait()
        pltpu.sync_copy(buf, acc_sh.at[sidx_v.at[ci]], add=True)

    fire(0, buf_a, sem_a)
    fire(1, buf_b, sem_b)

    def pair_body(p, carry):
        ci = 2 * p
        drain_scatter(ci, buf_a, sem_a)
        fire(ci + 2, buf_a, sem_a)
        drain_scatter(ci + 1, buf_b, sem_b)
        fire(ci + 3, buf_b, sem_b)
        return carry

    lax.fori_loop(0, NPAIR - 1, pair_body, 0)

    ci = NCHUNK - 2
    drain_scatter(ci, buf_a, sem_a)
    drain_scatter(ci + 1, buf_b, sem_b)

    plsc.subcore_barrier()
    pltpu.sync_copy(acc_sh.at[pl.ds(abase, BPW)],
                    out_hbm.at[pl.ds(wid * BPW, BPW)])


def _pool_slice(xi, tab):
    mesh = plsc.VectorSubcoreMesh(core_axis_name="c", subcore_axis_name="s")
    kern = pl.kernel(
        _pool_body,
        out_type=jax.ShapeDtypeStruct((B, C), jnp.float32),
        mesh=mesh,
        compiler_params=pltpu.CompilerParams(use_tc_tiling_on_sc=True),
        scratch_types=[
            pltpu.VMEM((NCHUNK * CROWS,), jnp.int32),
            pltpu.VMEM((NCHUNK, CROWS), jnp.int32),
            pltpu.VMEM((CROWS, C), jnp.float32),
            pltpu.VMEM((CROWS, C), jnp.float32),
            pltpu.VMEM_SHARED((B // 2, C), jnp.float32),
            pltpu.SemaphoreType.DMA,
            pltpu.SemaphoreType.DMA,
        ],
    )
    zeros = jnp.zeros((BPW, C), jnp.float32)
    return kern(xi, zeros, _SIDX, tab)


def _mlp_body(x0_ref, x1_ref, x2_ref, w1a_ref, w1b_ref, w1c_ref, b1_ref,
              w2_ref, b2_ref, w3_ref, b3_ref, o_ref):
    h = (jnp.dot(x0_ref[...], w1a_ref[...], preferred_element_type=jnp.float32)
         + jnp.dot(x1_ref[...], w1b_ref[...],
                   preferred_element_type=jnp.float32)
         + jnp.dot(x2_ref[...], w1c_ref[...],
                   preferred_element_type=jnp.float32))
    h = h * (1.0 / L) + b1_ref[...]
    h = jnp.where(h >= 0, h, 0.01 * h)
    h = jnp.dot(h, w2_ref[...], preferred_element_type=jnp.float32)
    h = h + b2_ref[...]
    h = jnp.where(h >= 0, h, 0.01 * h)
    h = jnp.dot(h, w3_ref[...], preferred_element_type=jnp.float32)
    o_ref[...] = h + b3_ref[...]


def _mlp(x0, x1, x2, w1a, w1b, w1c, b1, W2, b2, W3, b3):
    blk = 512
    xspec = pl.BlockSpec((blk, C), lambda i: (i, 0))
    w1spec = pl.BlockSpec((C, D), lambda i: (0, 0))
    wspec = pl.BlockSpec((D, D), lambda i: (0, 0))
    bspec = pl.BlockSpec((1, D), lambda i: (0, 0))
    return pl.pallas_call(
        _mlp_body,
        grid=(B // blk,),
        in_specs=[xspec, xspec, xspec, w1spec, w1spec, w1spec,
                  bspec, wspec, bspec, wspec, bspec],
        out_specs=pl.BlockSpec((blk, D), lambda i: (i, 0)),
        out_shape=jax.ShapeDtypeStruct((B, D), jnp.float32),
    )(x0, x1, x2, w1a, w1b, w1c, b1.reshape(1, D), W2, b2.reshape(1, D),
      W3, b3.reshape(1, D))


def kernel(x, table, W1, b1, W2, b2, W3, b3):
    xi = x.astype(jnp.int32).reshape(NW * NCHUNK * CROWS)
    t0 = table[:, 0:C]
    t1 = table[:, C:2 * C]
    t2 = table[:, D - C:D]
    w1a = W1[0:C]
    w1b = W1[C:2 * C]
    w1c = jnp.concatenate(
        [jnp.zeros((2 * C - (D - C), D), jnp.float32), W1[2 * C:D]], axis=0)
    x0 = _pool_slice(xi, t0)
    x1 = _pool_slice(xi, t1)
    x2 = _pool_slice(xi, t2)
    return _mlp(x0, x1, x2, w1a, w1b, w1c, b1, W2, b2, W3, b3)
